# baseline (speedup 1.0000x reference)
import functools

import jax
import jax.numpy as jnp
from jax import lax
from jax.experimental import pallas as pl
from jax.experimental.pallas import tpu as pltpu
from jax.experimental.pallas import tpu_sc as plsc

_NC = 2
_NS = 16
_NW = _NC * _NS
_CHUNK = 200
_GW = 100
_TBLK = 10000

def _pos_kernel(w_ref, b_ref, p_ref):
    S, D = p_ref.shape
    pos = lax.broadcasted_iota(jnp.int32, (S, D), 0).astype(jnp.float32)
    col = lax.broadcasted_iota(jnp.int32, (S, D), 1)
    two_i = ((col // 2) * 2).astype(jnp.float32)
    inv_div = jnp.exp(two_i * (-jnp.log(10000.0) / D))
    ang = pos * inv_div
    enc = jnp.where(col % 2 == 0, jnp.sin(ang), jnp.cos(ang))
    p_ref[...] = jnp.exp(2.0 * (
        jnp.dot(enc, w_ref[...], preferred_element_type=jnp.float32) + b_ref[...]
    ))


def _transform_kernel(tab_ref, w_ref, t2_ref):
    t16 = tab_ref[...].astype(jnp.bfloat16)
    w16 = (w_ref[...] * 2.0).astype(jnp.bfloat16)
    t2_ref[...] = jnp.exp(jnp.dot(t16, w16, preferred_element_type=jnp.float32))


def _tanh_from_exps(et, ep):
    r = 1.0 / (et * ep + 1.0)
    return 1.0 - (r + r)


def _sc_fused(t2, idx2d, p):
    n_idx_rows = idx2d.shape[0]
    d = t2.shape[1]
    n = n_idx_rows * _GW
    chunks_per_w = n // (_NW * _CHUNK)
    idx_rows_per_w = n_idx_rows // _NW
    mesh = plsc.VectorSubcoreMesh(core_axis_name="c", subcore_axis_name="s")

    @functools.partial(
        pl.kernel,
        out_type=jax.ShapeDtypeStruct((n, d), jnp.float32),
        mesh=mesh,
        scratch_types=[
            pltpu.VMEM((idx_rows_per_w, _GW), jnp.int32),
            pltpu.VMEM((_CHUNK, d), jnp.float32),
            pltpu.VMEM((3, _CHUNK, d), jnp.float32),
            pltpu.SemaphoreType.DMA,
            pltpu.SemaphoreType.DMA,
            pltpu.SemaphoreType.DMA,
            pltpu.SemaphoreType.DMA,
            pltpu.SemaphoreType.DMA,
            pltpu.SemaphoreType.DMA,
        ],
    )
    def kern(t2_hbm, idx_hbm, p_hbm, out_hbm, idx_v, p_v, rows_v, g0, g1, g2,
             w0, w1, w2):
        gs = (g0, g1, g2)
        ws = (w0, w1, w2)
        wid = lax.axis_index("s") * _NC + lax.axis_index("c")
        pltpu.sync_copy(idx_hbm.at[pl.ds(wid * idx_rows_per_w, idx_rows_per_w)],
                        idx_v)
        pltpu.sync_copy(p_hbm, p_v)
        row0 = wid * (chunks_per_w * _CHUNK)

        def gather_parts(q, b):
            return (
                (t2_hbm.at[idx_v.at[2 * q]], rows_v.at[b, pl.ds(0, _GW)]),
                (t2_hbm.at[idx_v.at[2 * q + 1]], rows_v.at[b, pl.ds(_GW, _GW)]),
            )

        def start_gather(q, b):
            for src, dst in gather_parts(q, b):
                pltpu.async_copy(src, dst, gs[b])

        def wait_gather(q, b):
            for src, dst in gather_parts(q, b):
                pltpu.make_async_copy(src, dst, gs[b]).wait()

        def out_slice(q):
            return out_hbm.at[pl.ds(row0 + q * _CHUNK, _CHUNK)]

        def start_write(q, b):
            pltpu.async_copy(rows_v.at[b], out_slice(q), ws[b])

        def wait_write(q, b):
            pltpu.make_async_copy(rows_v.at[b], out_slice(q), ws[b]).wait()

        def compute(b):
            buf = rows_v.at[b]

            @plsc.parallel_loop(0, _CHUNK, unroll=8)
            def _(r):
                for k in range(d // 16):
                    sl = pl.ds(k * 16, 16)
                    buf[r, sl] = _tanh_from_exps(buf[r, sl], p_v[r, sl])

        def chunk_body(q, b, wait_prev_write=True, prefetch=True):
            wait_gather(q, b)
            compute(b)
            start_write(q, b)
            if prefetch:
                b2 = (b + 2) % 3
                if wait_prev_write:
                    wait_write(q - 1, b2)
                start_gather(q + 2, b2)

        start_gather(0, 0)
        start_gather(1, 1)
        chunk_body(0, 0, wait_prev_write=False)
        chunk_body(1, 1)

        @pl.loop(0, (chunks_per_w - 5) // 3)
        def _(g):
            q = 3 * g + 2
            for b in range(3):
                chunk_body(q + b, (2 + b) % 3)

        chunk_body(chunks_per_w - 3, (chunks_per_w - 3) % 3)
        chunk_body(chunks_per_w - 2, (chunks_per_w - 2) % 3, prefetch=False)
        chunk_body(chunks_per_w - 1, (chunks_per_w - 1) % 3, prefetch=False)
        for q in (chunks_per_w - 3, chunks_per_w - 2, chunks_per_w - 1):
            wait_write(q, q % 3)

    return kern(t2, idx2d, p)


def kernel(x, tok_table, cat_tok_table, W_enc, b_enc):
    del cat_tok_table
    B, S = x.shape
    V, D = tok_table.shape
    N = B * S
    idx2d = x.reshape(N // _GW, _GW).astype(jnp.int32)
    W_top = W_enc[:D]
    W_bot = W_enc[D:]

    P = pl.pallas_call(
        _pos_kernel,
        out_shape=jax.ShapeDtypeStruct((S, D), jnp.float32),
    )(W_bot, b_enc.reshape(1, D))

    T2 = pl.pallas_call(
        _transform_kernel,
        grid=(V // _TBLK,),
        in_specs=[
            pl.BlockSpec((_TBLK, D), lambda i: (i, 0)),
            pl.BlockSpec((D, D), lambda i: (0, 0)),
        ],
        out_specs=pl.BlockSpec((_TBLK, D), lambda i: (i, 0)),
        out_shape=jax.ShapeDtypeStruct((V, D), jnp.float32),
    )(tok_table, W_top)

    out = _sc_fused(T2, idx2d, P)
    return out.reshape(B, S, D)


# speedup vs baseline: 1.0038x; 1.0038x over previous
import functools

import jax
import jax.numpy as jnp
from jax import lax
from jax.experimental import pallas as pl
from jax.experimental.pallas import tpu as pltpu
from jax.experimental.pallas import tpu_sc as plsc

_NC = 2
_NS = 16
_NW = _NC * _NS
_CHUNK = 200
_GW = 100
_TBLK = 10000

def _pos_kernel(w_ref, b_ref, p_ref):
    S, D = p_ref.shape
    pos = lax.broadcasted_iota(jnp.int32, (S, D), 0).astype(jnp.float32)
    col = lax.broadcasted_iota(jnp.int32, (S, D), 1)
    two_i = ((col // 2) * 2).astype(jnp.float32)
    inv_div = jnp.exp(two_i * (-jnp.log(10000.0) / D))
    ang = pos * inv_div
    enc = jnp.where(col % 2 == 0, jnp.sin(ang), jnp.cos(ang))
    p_ref[...] = jnp.exp(2.0 * (
        jnp.dot(enc, w_ref[...], preferred_element_type=jnp.float32) + b_ref[...]
    ))


def _transform_kernel(tab_ref, w_ref, t2_ref):
    t16 = tab_ref[...].astype(jnp.bfloat16)
    w16 = (w_ref[...] * 2.0).astype(jnp.bfloat16)
    t2_ref[...] = jnp.exp(jnp.dot(t16, w16, preferred_element_type=jnp.float32))


def _tanh_from_exps(et, ep):
    r = 1.0 / (et * ep + 1.0)
    return 1.0 - (r + r)


def _sc_fused(t2, idx2d, p):
    n_idx_rows = idx2d.shape[0]
    d = t2.shape[1]
    n = n_idx_rows * _CHUNK
    chunks_per_w = n // (_NW * _CHUNK)
    idx_rows_per_w = n_idx_rows // _NW
    mesh = plsc.VectorSubcoreMesh(core_axis_name="c", subcore_axis_name="s")

    @functools.partial(
        pl.kernel,
        out_type=jax.ShapeDtypeStruct((n, d), jnp.float32),
        mesh=mesh,
        scratch_types=[
            pltpu.VMEM((idx_rows_per_w, _CHUNK), jnp.int32),
            pltpu.VMEM((_CHUNK, d), jnp.float32),
            pltpu.VMEM((3, _CHUNK, d), jnp.float32),
            pltpu.SemaphoreType.DMA,
            pltpu.SemaphoreType.DMA,
            pltpu.SemaphoreType.DMA,
            pltpu.SemaphoreType.DMA,
            pltpu.SemaphoreType.DMA,
            pltpu.SemaphoreType.DMA,
        ],
    )
    def kern(t2_hbm, idx_hbm, p_hbm, out_hbm, idx_v, p_v, rows_v, g0, g1, g2,
             w0, w1, w2):
        gs = (g0, g1, g2)
        ws = (w0, w1, w2)
        wid = lax.axis_index("s") * _NC + lax.axis_index("c")
        pltpu.sync_copy(idx_hbm.at[pl.ds(wid * idx_rows_per_w, idx_rows_per_w)],
                        idx_v)
        pltpu.sync_copy(p_hbm, p_v)
        row0 = wid * (chunks_per_w * _CHUNK)

        def gather_parts(q, b):
            return (
                (t2_hbm.at[idx_v.at[q, pl.ds(0, 128)]],
                 rows_v.at[b, pl.ds(0, 128)]),
                (t2_hbm.at[idx_v.at[q, pl.ds(128, _CHUNK - 128)]],
                 rows_v.at[b, pl.ds(128, _CHUNK - 128)]),
            )

        def start_gather(q, b):
            for src, dst in gather_parts(q, b):
                pltpu.async_copy(src, dst, gs[b])

        def wait_gather(q, b):
            for src, dst in gather_parts(q, b):
                pltpu.make_async_copy(src, dst, gs[b]).wait()

        def out_slice(q):
            return out_hbm.at[pl.ds(row0 + q * _CHUNK, _CHUNK)]

        def start_write(q, b):
            pltpu.async_copy(rows_v.at[b], out_slice(q), ws[b])

        def wait_write(q, b):
            pltpu.make_async_copy(rows_v.at[b], out_slice(q), ws[b]).wait()

        def compute(b):
            buf = rows_v.at[b]

            @plsc.parallel_loop(0, _CHUNK, unroll=4)
            def _(r):
                for k in range(d // 16):
                    sl = pl.ds(k * 16, 16)
                    buf[r, sl] = _tanh_from_exps(buf[r, sl], p_v[r, sl])

        def chunk_body(q, b, wait_prev_write=True, prefetch=True):
            wait_gather(q, b)
            compute(b)
            start_write(q, b)
            if prefetch:
                b2 = (b + 2) % 3
                if wait_prev_write:
                    wait_write(q - 1, b2)
                start_gather(q + 2, b2)

        start_gather(0, 0)
        start_gather(1, 1)
        chunk_body(0, 0, wait_prev_write=False)
        chunk_body(1, 1)

        @pl.loop(0, (chunks_per_w - 5) // 3)
        def _(g):
            q = 3 * g + 2
            for b in range(3):
                chunk_body(q + b, (2 + b) % 3)

        chunk_body(chunks_per_w - 3, (chunks_per_w - 3) % 3)
        chunk_body(chunks_per_w - 2, (chunks_per_w - 2) % 3, prefetch=False)
        chunk_body(chunks_per_w - 1, (chunks_per_w - 1) % 3, prefetch=False)
        for q in (chunks_per_w - 3, chunks_per_w - 2, chunks_per_w - 1):
            wait_write(q, q % 3)

    return kern(t2, idx2d, p)


def kernel(x, tok_table, cat_tok_table, W_enc, b_enc):
    del cat_tok_table
    B, S = x.shape
    V, D = tok_table.shape
    N = B * S
    idx2d = x.astype(jnp.int32)
    W_top = W_enc[:D]
    W_bot = W_enc[D:]

    P = pl.pallas_call(
        _pos_kernel,
        out_shape=jax.ShapeDtypeStruct((S, D), jnp.float32),
    )(W_bot, b_enc.reshape(1, D))

    T2 = pl.pallas_call(
        _transform_kernel,
        grid=(V // _TBLK,),
        in_specs=[
            pl.BlockSpec((_TBLK, D), lambda i: (i, 0)),
            pl.BlockSpec((D, D), lambda i: (0, 0)),
        ],
        out_specs=pl.BlockSpec((_TBLK, D), lambda i: (i, 0)),
        out_shape=jax.ShapeDtypeStruct((V, D), jnp.float32),
    )(tok_table, W_top)

    out = _sc_fused(T2, idx2d, P)
    return out.reshape(B, S, D)


# speedup vs baseline: 1.0283x; 1.0243x over previous
import functools

import jax
import jax.numpy as jnp
from jax import lax
from jax.experimental import pallas as pl
from jax.experimental.pallas import tpu as pltpu
from jax.experimental.pallas import tpu_sc as plsc

_NC = 2
_NS = 16
_NW = _NC * _NS
_CHUNK = 200
_GW = 100
_TBLK = 20000

def _pos_kernel(w_ref, b_ref, p_ref):
    S, D = p_ref.shape
    pos = lax.broadcasted_iota(jnp.int32, (S, D), 0).astype(jnp.float32)
    col = lax.broadcasted_iota(jnp.int32, (S, D), 1)
    two_i = ((col // 2) * 2).astype(jnp.float32)
    inv_div = jnp.exp(two_i * (-jnp.log(10000.0) / D))
    ang = pos * inv_div
    enc = jnp.where(col % 2 == 0, jnp.sin(ang), jnp.cos(ang))
    p_ref[...] = jnp.exp(2.0 * (
        jnp.dot(enc, w_ref[...], preferred_element_type=jnp.float32) + b_ref[...]
    ))


def _transform_kernel(tab_ref, w_ref, t2_ref):
    t16 = tab_ref[...].astype(jnp.bfloat16)
    w16 = (w_ref[...] * 2.0).astype(jnp.bfloat16)
    t2_ref[...] = jnp.exp(jnp.dot(t16, w16, preferred_element_type=jnp.float32))


def _tanh_from_exps(et, ep):
    r = 1.0 / (et * ep + 1.0)
    return 1.0 - (r + r)


def _sc_fused(t2, idx2d, p):
    n_idx_rows = idx2d.shape[0]
    d = t2.shape[1]
    n = n_idx_rows * _CHUNK
    chunks_per_w = n // (_NW * _CHUNK)
    idx_rows_per_w = n_idx_rows // _NW
    mesh = plsc.VectorSubcoreMesh(core_axis_name="c", subcore_axis_name="s")

    @functools.partial(
        pl.kernel,
        out_type=jax.ShapeDtypeStruct((n, d), jnp.float32),
        mesh=mesh,
        scratch_types=[
            pltpu.VMEM((idx_rows_per_w, _CHUNK), jnp.int32),
            pltpu.VMEM((_CHUNK, d), jnp.float32),
            pltpu.VMEM((3, _CHUNK, d), jnp.float32),
            pltpu.SemaphoreType.DMA,
            pltpu.SemaphoreType.DMA,
            pltpu.SemaphoreType.DMA,
            pltpu.SemaphoreType.DMA,
            pltpu.SemaphoreType.DMA,
            pltpu.SemaphoreType.DMA,
        ],
    )
    def kern(t2_hbm, idx_hbm, p_hbm, out_hbm, idx_v, p_v, rows_v, g0, g1, g2,
             w0, w1, w2):
        gs = (g0, g1, g2)
        ws = (w0, w1, w2)
        wid = lax.axis_index("s") * _NC + lax.axis_index("c")
        pltpu.sync_copy(idx_hbm.at[pl.ds(wid * idx_rows_per_w, idx_rows_per_w)],
                        idx_v)
        pltpu.sync_copy(p_hbm, p_v)
        row0 = wid * (chunks_per_w * _CHUNK)

        def gather_parts(q, b):
            return (
                (t2_hbm.at[idx_v.at[q, pl.ds(0, 128)]],
                 rows_v.at[b, pl.ds(0, 128)]),
                (t2_hbm.at[idx_v.at[q, pl.ds(128, _CHUNK - 128)]],
                 rows_v.at[b, pl.ds(128, _CHUNK - 128)]),
            )

        def start_gather(q, b):
            for src, dst in gather_parts(q, b):
                pltpu.async_copy(src, dst, gs[b])

        def wait_gather(q, b):
            for src, dst in gather_parts(q, b):
                pltpu.make_async_copy(src, dst, gs[b]).wait()

        def out_slice(q):
            return out_hbm.at[pl.ds(row0 + q * _CHUNK, _CHUNK)]

        def start_write(q, b):
            pltpu.async_copy(rows_v.at[b], out_slice(q), ws[b])

        def wait_write(q, b):
            pltpu.make_async_copy(rows_v.at[b], out_slice(q), ws[b]).wait()

        def compute(b):
            buf = rows_v.at[b]

            @plsc.parallel_loop(0, _CHUNK, unroll=4)
            def _(r):
                for k in range(d // 16):
                    sl = pl.ds(k * 16, 16)
                    buf[r, sl] = _tanh_from_exps(buf[r, sl], p_v[r, sl])

        def chunk_body(q, b, wait_prev_write=True, prefetch=True):
            wait_gather(q, b)
            compute(b)
            start_write(q, b)
            if prefetch:
                b2 = (b + 2) % 3
                if wait_prev_write:
                    wait_write(q - 1, b2)
                start_gather(q + 2, b2)

        start_gather(0, 0)
        start_gather(1, 1)
        chunk_body(0, 0, wait_prev_write=False)
        chunk_body(1, 1)

        @pl.loop(0, (chunks_per_w - 5) // 3)
        def _(g):
            q = 3 * g + 2
            for b in range(3):
                chunk_body(q + b, (2 + b) % 3)

        chunk_body(chunks_per_w - 3, (chunks_per_w - 3) % 3)
        chunk_body(chunks_per_w - 2, (chunks_per_w - 2) % 3, prefetch=False)
        chunk_body(chunks_per_w - 1, (chunks_per_w - 1) % 3, prefetch=False)
        for q in (chunks_per_w - 3, chunks_per_w - 2, chunks_per_w - 1):
            wait_write(q, q % 3)

    return kern(t2, idx2d, p)


def kernel(x, tok_table, cat_tok_table, W_enc, b_enc):
    del cat_tok_table
    B, S = x.shape
    V, D = tok_table.shape
    N = B * S
    idx2d = x.astype(jnp.int32)
    W_top = W_enc[:D]
    W_bot = W_enc[D:]

    P = pl.pallas_call(
        _pos_kernel,
        out_shape=jax.ShapeDtypeStruct((S, D), jnp.float32),
    )(W_bot, b_enc.reshape(1, D))

    T2 = pl.pallas_call(
        _transform_kernel,
        grid=(V // _TBLK,),
        in_specs=[
            pl.BlockSpec((_TBLK, D), lambda i: (i, 0)),
            pl.BlockSpec((D, D), lambda i: (0, 0)),
        ],
        out_specs=pl.BlockSpec((_TBLK, D), lambda i: (i, 0)),
        out_shape=jax.ShapeDtypeStruct((V, D), jnp.float32),
    )(tok_table, W_top)

    out = _sc_fused(T2, idx2d, P)
    return out.reshape(B, S, D)
